# Initial kernel scaffold; baseline (speedup 1.0000x reference)
import jax
import jax.numpy as jnp
from jax.experimental import pallas as pl


def kernel(word_inputs, word_seq_lengths, embedding_weight):
    raise NotImplementedError("write your pallas kernel here")


# speedup vs baseline: 3.3338x; 3.3338x over previous
import functools

import jax
import jax.numpy as jnp
from jax import lax
from jax.experimental import pallas as pl
from jax.experimental.pallas import tpu as pltpu
from jax.experimental.pallas import tpu_sc as plsc

VOCAB = 100000
EMB_DIM = 128
BATCH = 4096
SEQ_LEN = 50

NC = 2
NS = 16
NW = NC * NS
B = BATCH * SEQ_LEN
BPW = B // NW
CHUNK = 128
NCHUNK = BPW // CHUNK
NBUF = 2

_mesh = plsc.VectorSubcoreMesh(core_axis_name="c", subcore_axis_name="s")


@functools.partial(
    pl.kernel,
    mesh=_mesh,
    out_type=jax.ShapeDtypeStruct((B, EMB_DIM), jnp.float32),
    scratch_types=[
        pltpu.VMEM((NCHUNK, CHUNK), jnp.int32),
        pltpu.VMEM((CHUNK, EMB_DIM), jnp.float32),
        pltpu.VMEM((CHUNK, EMB_DIM), jnp.float32),
        pltpu.SemaphoreType.DMA,
        pltpu.SemaphoreType.DMA,
    ],
)
def _sc_gather(idx_hbm, table_hbm, out_hbm, idx_v, rows0, rows1, sem0, sem1):
    wid = lax.axis_index("s") * NC + lax.axis_index("c")
    base = wid * BPW
    rows = (rows0, rows1)
    sems = (sem0, sem1)

    pltpu.sync_copy(idx_hbm.at[wid], idx_v)

    for b in range(NBUF):
        pltpu.async_copy(table_hbm.at[idx_v.at[b]], rows[b], sems[b])

    def body(i, carry):
        for b in range(NBUF):
            j = i * NBUF + b
            pltpu.make_async_copy(
                table_hbm.at[idx_v.at[j]], rows[b], sems[b]
            ).wait()
            pltpu.sync_copy(rows[b], out_hbm.at[pl.ds(base + j * CHUNK, CHUNK)])

            @pl.when(j + NBUF < NCHUNK)
            def _():
                pltpu.async_copy(
                    table_hbm.at[idx_v.at[j + NBUF]], rows[b], sems[b]
                )

        return carry

    lax.fori_loop(0, NCHUNK // NBUF, body, 0)


def kernel(word_inputs, word_seq_lengths, embedding_weight):
    del word_seq_lengths
    idx = word_inputs.reshape(NW, NCHUNK, CHUNK)
    out = _sc_gather(idx, embedding_weight)
    return out.reshape(BATCH, SEQ_LEN, EMB_DIM)


# speedup vs baseline: 3.3495x; 1.0047x over previous
import functools

import jax
import jax.numpy as jnp
from jax import lax
from jax.experimental import pallas as pl
from jax.experimental.pallas import tpu as pltpu
from jax.experimental.pallas import tpu_sc as plsc

VOCAB = 100000
EMB_DIM = 128
BATCH = 4096
SEQ_LEN = 50

NC = 2
NS = 16
NW = NC * NS
B = BATCH * SEQ_LEN
BPW = B // NW
CHUNK = 128
NCHUNK = BPW // CHUNK
NBUF = 5
assert NCHUNK % NBUF == 0

_mesh = plsc.VectorSubcoreMesh(core_axis_name="c", subcore_axis_name="s")


@functools.partial(
    pl.kernel,
    mesh=_mesh,
    out_type=jax.ShapeDtypeStruct((B, EMB_DIM), jnp.float32),
    scratch_types=(
        [pltpu.VMEM((NCHUNK, CHUNK), jnp.int32)]
        + [pltpu.VMEM((CHUNK, EMB_DIM), jnp.float32)] * NBUF
        + [pltpu.SemaphoreType.DMA] * NBUF
    ),
)
def _sc_gather(idx_hbm, table_hbm, out_hbm, idx_v, *bufs):
    rows = bufs[:NBUF]
    sems = bufs[NBUF:]
    wid = lax.axis_index("s") * NC + lax.axis_index("c")
    base = wid * BPW

    pltpu.sync_copy(idx_hbm.at[wid], idx_v)

    for b in range(NBUF):
        pltpu.async_copy(table_hbm.at[idx_v.at[b]], rows[b], sems[b])

    def body(i, carry):
        for b in range(NBUF):
            j = i * NBUF + b
            pltpu.make_async_copy(
                table_hbm.at[idx_v.at[j]], rows[b], sems[b]
            ).wait()
            pltpu.sync_copy(rows[b], out_hbm.at[pl.ds(base + j * CHUNK, CHUNK)])

            @pl.when(j + NBUF < NCHUNK)
            def _():
                pltpu.async_copy(
                    table_hbm.at[idx_v.at[j + NBUF]], rows[b], sems[b]
                )

        return carry

    lax.fori_loop(0, NCHUNK // NBUF, body, 0)


def kernel(word_inputs, word_seq_lengths, embedding_weight):
    del word_seq_lengths
    idx = word_inputs.reshape(NW, NCHUNK, CHUNK)
    out = _sc_gather(idx, embedding_weight)
    return out.reshape(BATCH, SEQ_LEN, EMB_DIM)


# speedup vs baseline: 5.9391x; 1.7731x over previous
import functools

import jax
import jax.numpy as jnp
from jax import lax
from jax.experimental import pallas as pl
from jax.experimental.pallas import tpu as pltpu
from jax.experimental.pallas import tpu_sc as plsc

VOCAB = 100000
EMB_DIM = 128
BATCH = 4096
SEQ_LEN = 50

NC = 2
NS = 16
NW = NC * NS
NBPW = BATCH // NW
NBUF = 4
assert NBPW % NBUF == 0

_mesh = plsc.VectorSubcoreMesh(core_axis_name="c", subcore_axis_name="s")


@functools.partial(
    pl.kernel,
    mesh=_mesh,
    out_type=jax.ShapeDtypeStruct((BATCH, SEQ_LEN, EMB_DIM), jnp.float32),
    scratch_types=(
        [pltpu.VMEM((NBPW, SEQ_LEN), jnp.int32)]
        + [pltpu.VMEM((SEQ_LEN, EMB_DIM), jnp.float32)] * NBUF
        + [pltpu.SemaphoreType.DMA] * NBUF
    ),
    compiler_params=pltpu.CompilerParams(use_tc_tiling_on_sc=True),
)
def _sc_gather(idx_hbm, table_hbm, out_hbm, idx_v, *bufs):
    rows = bufs[:NBUF]
    sems = bufs[NBUF:]
    wid = lax.axis_index("s") * NC + lax.axis_index("c")
    base = wid * NBPW

    pltpu.sync_copy(idx_hbm.at[pl.ds(base, NBPW)], idx_v)

    for b in range(NBUF):
        pltpu.async_copy(table_hbm.at[idx_v.at[b]], rows[b], sems[b])

    def body(i, carry):
        for b in range(NBUF):
            j = i * NBUF + b
            pltpu.make_async_copy(
                table_hbm.at[idx_v.at[j]], rows[b], sems[b]
            ).wait()
            pltpu.sync_copy(rows[b], out_hbm.at[base + j])

            @pl.when(j + NBUF < NBPW)
            def _():
                pltpu.async_copy(
                    table_hbm.at[idx_v.at[j + NBUF]], rows[b], sems[b]
                )

        return carry

    lax.fori_loop(0, NBPW // NBUF, body, 0)


def kernel(word_inputs, word_seq_lengths, embedding_weight):
    del word_seq_lengths
    return _sc_gather(word_inputs, embedding_weight)
